# Initial kernel scaffold; baseline (speedup 1.0000x reference)
import jax
import jax.numpy as jnp
from jax.experimental import pallas as pl


def kernel(pred, targ, batch):
    raise NotImplementedError("write your pallas kernel here")


# speedup vs baseline: 3.7074x; 3.7074x over previous
import jax
import jax.numpy as jnp
from jax.experimental import pallas as pl
from jax.experimental.pallas import tpu as pltpu

_N = 320000
_D = 128
_S = 512
_R = 2560
_G = _N // _R


def _body(batch_ref, pred_ref, targ_ref, out_ref, acc_ref):
    i = pl.program_id(0)

    @pl.when(i == 0)
    def _init():
        acc_ref[...] = jnp.zeros_like(acc_ref)

    d = pred_ref[...] - targ_ref[...]
    loss = jnp.sum(d * d, axis=1) * (1.0 / _D)
    ids = batch_ref[0, 0, :]
    onehot = (ids[:, None] == jax.lax.broadcasted_iota(jnp.int32, (1, _S), 1)
              ).astype(jnp.float32)
    stacked = jnp.stack([loss, jnp.ones_like(loss)], axis=0)
    part = jax.lax.dot_general(
        stacked, onehot, (((1,), (0,)), ((), ())),
        preferred_element_type=jnp.float32)
    acc_ref[...] += part

    @pl.when(i == _G - 1)
    def _fin():
        out_ref[...] = acc_ref[0, :] / acc_ref[1, :]


@jax.jit
def kernel(pred, targ, batch):
    batch3 = batch.reshape(_G, 1, _R)
    return pl.pallas_call(
        _body,
        grid=(_G,),
        in_specs=[
            pl.BlockSpec((1, 1, _R), lambda i: (i, 0, 0)),
            pl.BlockSpec((_R, _D), lambda i: (i, 0)),
            pl.BlockSpec((_R, _D), lambda i: (i, 0)),
        ],
        out_specs=pl.BlockSpec((_S,), lambda i: (0,)),
        out_shape=jax.ShapeDtypeStruct((_S,), jnp.float32),
        scratch_shapes=[pltpu.VMEM((2, _S), jnp.float32)],
    )(batch3, pred, targ)


# speedup vs baseline: 4.2203x; 1.1384x over previous
import functools

import jax
import jax.numpy as jnp
from jax import lax
from jax.experimental import pallas as pl
from jax.experimental.pallas import tpu as pltpu
from jax.experimental.pallas import tpu_sc as plsc

_N = 320000
_D = 128
_S = 512
_R = 2560
_G = _N // _R

_NC = 2
_NS = 16
_NW = _NC * _NS
_L = 16
_CHUNK = _N // _NW
_ITERS = _CHUNK // _L
_BINS = _S * _L
_SEG_PER_W = _S // _NW


_NR = _N // _D
_BR = _R // _D


def _tc_loss_body(pred_ref, targ_ref, out_ref):
    i = pl.program_id(0)
    d = pred_ref[...] - targ_ref[...]
    s = jnp.sum(d * d, axis=2)
    out_ref[pl.ds(i * _BR, _BR), :] = s * (1.0 / _D)


def _tc_loss(pred, targ):
    pred3 = pred.reshape(_NR, _D, _D)
    targ3 = targ.reshape(_NR, _D, _D)
    out = pl.pallas_call(
        _tc_loss_body,
        grid=(_G,),
        in_specs=[
            pl.BlockSpec((_BR, _D, _D), lambda i: (i, 0, 0)),
            pl.BlockSpec((_BR, _D, _D), lambda i: (i, 0, 0)),
        ],
        out_specs=pl.BlockSpec((_NR, _D), lambda i: (0, 0)),
        out_shape=jax.ShapeDtypeStruct((_NR, _D), jnp.float32),
    )(pred3, targ3)
    return out.reshape(_N)


def _sc_phase1_body(loss_hbm, batch_hbm, sums_hbm, counts_hbm,
                    loss_v, ids_v, bins_v, cnt_v):
    wid = lax.axis_index("s") * _NC + lax.axis_index("c")
    base = wid * _CHUNK
    pltpu.sync_copy(loss_hbm.at[pl.ds(base, _CHUNK)], loss_v)
    pltpu.sync_copy(batch_hbm.at[pl.ds(base, _CHUNK)], ids_v)

    zeros = jnp.zeros((_L,), jnp.float32)

    def zero_body(j, carry):
        bins_v[pl.ds(j * _L, _L)] = zeros
        cnt_v[pl.ds(j * _L, _L)] = zeros
        return carry

    lax.fori_loop(0, _S, zero_body, 0)

    lane = lax.iota(jnp.int32, _L)
    ones = jnp.ones((_L,), jnp.float32)

    def body(i, carry):
        l = loss_v[pl.ds(i * _L, _L)]
        s = ids_v[pl.ds(i * _L, _L)]
        idx = s * _L + lane
        plsc.addupdate_scatter(bins_v, [idx], l)
        plsc.addupdate_scatter(cnt_v, [idx], ones)
        return carry

    lax.fori_loop(0, _ITERS, body, 0)

    pltpu.sync_copy(bins_v, sums_hbm.at[wid])
    pltpu.sync_copy(cnt_v, counts_hbm.at[wid])


_COLS = _SEG_PER_W * _L


def _sc_phase2_body(sums_hbm, counts_hbm, out_hbm,
                    sums_v, cnts_v, acc_s, acc_c, out_v):
    wid = lax.axis_index("s") * _NC + lax.axis_index("c")
    col0 = wid * _COLS

    pltpu.sync_copy(sums_hbm.at[:, pl.ds(col0, _COLS)], sums_v)
    pltpu.sync_copy(counts_hbm.at[:, pl.ds(col0, _COLS)], cnts_v)

    zeros = jnp.zeros((_L,), jnp.float32)
    for j in range(_SEG_PER_W):
        acc_s[pl.ds(j * _L, _L)] = zeros
        acc_c[pl.ds(j * _L, _L)] = zeros

    def body(p, carry):
        for j in range(_SEG_PER_W):
            sl = pl.ds(j * _L, _L)
            acc_s[sl] = acc_s[sl] + sums_v[p, sl]
            acc_c[sl] = acc_c[sl] + cnts_v[p, sl]
        return carry

    lax.fori_loop(0, _NW, body, 0)

    seg16 = lax.iota(jnp.int32, _L) * _L
    tot_s = jnp.zeros((_L,), jnp.float32)
    tot_c = jnp.zeros((_L,), jnp.float32)
    for j in range(_L):
        tot_s = tot_s + plsc.load_gather(acc_s, [seg16 + j])
        tot_c = tot_c + plsc.load_gather(acc_c, [seg16 + j])

    out_v[...] = tot_s / tot_c
    pltpu.sync_copy(out_v, out_hbm.at[pl.ds(wid * _SEG_PER_W, _SEG_PER_W)])


_sc_cache = []


def _sc_kernels():
    if not _sc_cache:
        mesh = plsc.VectorSubcoreMesh(
            core_axis_name="c", subcore_axis_name="s",
            num_cores=_NC, num_subcores=_NS)
        params = pltpu.CompilerParams(needs_layout_passes=False)
        phase1 = pl.kernel(
            _sc_phase1_body,
            compiler_params=params,
            out_type=[
                jax.ShapeDtypeStruct((_NW, _BINS), jnp.float32),
                jax.ShapeDtypeStruct((_NW, _BINS), jnp.float32),
            ],
            mesh=mesh,
            scratch_types=[
                pltpu.VMEM((_CHUNK,), jnp.float32),
                pltpu.VMEM((_CHUNK,), jnp.int32),
                pltpu.VMEM((_BINS,), jnp.float32),
                pltpu.VMEM((_BINS,), jnp.float32),
            ],
        )
        phase2 = pl.kernel(
            _sc_phase2_body,
            compiler_params=params,
            out_type=jax.ShapeDtypeStruct((_S,), jnp.float32),
            mesh=mesh,
            scratch_types=[
                pltpu.VMEM((_NW, _COLS), jnp.float32),
                pltpu.VMEM((_NW, _COLS), jnp.float32),
                pltpu.VMEM((_COLS,), jnp.float32),
                pltpu.VMEM((_COLS,), jnp.float32),
                pltpu.VMEM((_L,), jnp.float32),
            ],
        )
        _sc_cache.append((phase1, phase2))
    return _sc_cache[0]


@jax.jit
def kernel(pred, targ, batch):
    phase1, phase2 = _sc_kernels()
    loss = _tc_loss(pred, targ)
    sums_p, counts_p = phase1(loss, batch)
    return phase2(sums_p, counts_p)
